# baseline (speedup 1.0000x reference)
import functools

import jax
import jax.numpy as jnp
from jax.experimental import pallas as pl

EPS = 1e-05
NUM_EXPERTS = 64
TOP_K = 8


def _router_block(x_ref, gw_ref, eb_ref, probs_ref, idx_ref, logits_ref):
    x = x_ref[...]
    mean = jnp.mean(x, axis=-1, keepdims=True)
    cx = x - mean
    var = jnp.mean(cx * cx, axis=-1, keepdims=True)
    xn = cx / jnp.sqrt(var + EPS)
    logits = jax.lax.dot_general(
        xn, gw_ref[...],
        dimension_numbers=(((1,), (1,)), ((), ())),
        preferred_element_type=jnp.float32,
    )
    logits = logits + eb_ref[...]
    logits_ref[...] = logits
    lt = logits.T
    lmax = jnp.max(lt, axis=0, keepdims=True)
    e = jnp.exp(lt - lmax)
    bt = lt.shape[1]
    iota = jax.lax.broadcasted_iota(jnp.int32, (NUM_EXPERTS, bt), 0)
    work = e
    vals = []
    idxs = []
    for _ in range(TOP_K):
        m = jnp.max(work, axis=0, keepdims=True)
        am = jnp.min(jnp.where(work == m, iota, NUM_EXPERTS),
                     axis=0, keepdims=True)
        vals.append(m)
        idxs.append(am)
        work = jnp.where(iota == am, -jnp.inf, work)
    top_e = jnp.concatenate(vals, axis=0)
    top_idx = jnp.concatenate(idxs, axis=0)
    s = jnp.sum(top_e, axis=0, keepdims=True)
    probs_ref[...] = (top_e / s).T
    idx_ref[...] = top_idx.T


@functools.partial(jax.jit, static_argnames=())
def kernel(hidden_states, ln_weight, ln_bias, gate_weight, expert_bias):
    B, S, H = hidden_states.shape
    T = B * S
    E = gate_weight.shape[0]
    x = hidden_states.reshape(T, H)
    gw = gate_weight * ln_weight[None, :]
    eb = (expert_bias + gate_weight @ ln_bias).reshape(1, E)

    BT = 1024
    grid = (T // BT,)

    probs, idx, logits = pl.pallas_call(
        _router_block,
        grid=grid,
        in_specs=[
            pl.BlockSpec((BT, H), lambda i: (i, 0)),
            pl.BlockSpec((E, H), lambda i: (0, 0)),
            pl.BlockSpec((1, E), lambda i: (0, 0)),
        ],
        out_specs=[
            pl.BlockSpec((BT, TOP_K), lambda i: (i, 0)),
            pl.BlockSpec((BT, TOP_K), lambda i: (i, 0)),
            pl.BlockSpec((BT, E), lambda i: (i, 0)),
        ],
        out_shape=[
            jax.ShapeDtypeStruct((T, TOP_K), jnp.float32),
            jax.ShapeDtypeStruct((T, TOP_K), jnp.int32),
            jax.ShapeDtypeStruct((T, E), jnp.float32),
        ],
    )(x, gw, eb)
    return probs, idx, logits
